# baseline (speedup 1.0000x reference)
import functools

import jax
import jax.numpy as jnp
from jax import lax
from jax.experimental import pallas as pl
from jax.experimental.pallas import tpu as pltpu
from jax.experimental.pallas import tpu_sc as plsc

LANES = 16
CHUNK = 128
NBUF = 2


def _fast_rsqrt(x):
    xi = lax.bitcast_convert_type(x, jnp.int32)
    yi = jnp.int32(0x5F3759DF) - lax.shift_right_arithmetic(xi, 1)
    y = lax.bitcast_convert_type(yi, jnp.float32)
    xh = x * jnp.float32(0.5)
    for _ in range(4):
        y = y * (jnp.float32(1.5) - xh * y * y)
    return y


def _make_kernel(B, L, D, NW):
    N = B * L
    assert N % (NW * CHUNK) == 0
    NCH = N // (NW * CHUNK)
    PER_W = NCH * CHUNK
    assert PER_W % L == 0
    NVREG = D // LANES
    POS_ROWS = 2 * L

    mesh = plsc.VectorSubcoreMesh(core_axis_name="c", subcore_axis_name="s")

    @functools.partial(
        pl.kernel,
        mesh=mesh,
        compiler_params=pltpu.CompilerParams(
            needs_layout_passes=False, use_tc_tiling_on_sc=False),
        out_type=jax.ShapeDtypeStruct((N, D), jnp.float32),
        scratch_types=[
            pltpu.VMEM((NCH, CHUNK), jnp.int32),
            pltpu.VMEM((POS_ROWS, D), jnp.float32),
            pltpu.VMEM((NBUF, CHUNK, D), jnp.float32),
            pltpu.VMEM((NBUF, CHUNK, D), jnp.float32),
            pltpu.VMEM((D,), jnp.float32),
            pltpu.VMEM((D,), jnp.float32),
            pltpu.SemaphoreType.DMA,
            pltpu.SemaphoreType.DMA,
            pltpu.SemaphoreType.DMA,
            pltpu.SemaphoreType.DMA,
        ],
    )
    def emb_ln(ids_hbm, tok_hbm, pos_hbm, gamma_hbm, beta_hbm, out_hbm,
               idx_v, pos_v, rows_v, outb_v, g_v, b_v,
               gsem0, gsem1, ssem0, ssem1):
        gsems = (gsem0, gsem1)
        ssems = (ssem0, ssem1)
        wid = lax.axis_index("s") * 2 + lax.axis_index("c")
        wbase = wid * PER_W

        pltpu.sync_copy(ids_hbm.at[wid], idx_v)
        pltpu.sync_copy(pos_hbm.at[pl.ds(0, L)], pos_v.at[pl.ds(0, L)])
        pltpu.sync_copy(pos_hbm.at[pl.ds(0, L)], pos_v.at[pl.ds(L, L)])
        pltpu.sync_copy(gamma_hbm, g_v)
        pltpu.sync_copy(beta_hbm, b_v)

        g_regs = [g_v[pl.ds(LANES * j, LANES)] for j in range(NVREG)]
        b_regs = [b_v[pl.ds(LANES * j, LANES)] for j in range(NVREG)]

        inv_d = jnp.float32(1.0 / D)
        eps = jnp.float32(1e-12)

        def gather_start(n, b):
            pltpu.async_copy(tok_hbm.at[idx_v.at[n]], rows_v.at[b], gsems[b])

        def gather_wait(n, b):
            pltpu.make_async_copy(
                tok_hbm.at[idx_v.at[n]], rows_v.at[b], gsems[b]
            ).wait()

        def store_start(n, b):
            pltpu.async_copy(
                outb_v.at[b], out_hbm.at[pl.ds(wbase + n * CHUNK, CHUNK)],
                ssems[b])

        def store_wait(n, b):
            pltpu.make_async_copy(
                outb_v.at[b], out_hbm.at[pl.ds(wbase + n * CHUNK, CHUNK)],
                ssems[b]).wait()

        def compute_chunk(n, b):
            p0 = lax.rem(n * CHUNK, L)

            def tok(i, carry):
                r = [rows_v[b, i, pl.ds(LANES * j, LANES)]
                     for j in range(NVREG)]
                p = [pos_v[p0 + i, pl.ds(LANES * j, LANES)]
                     for j in range(NVREG)]
                e = [r[j] + p[j] for j in range(NVREG)]
                s = (e[0] + e[1]) + (e[2] + e[3])
                sq = (e[0] * e[0] + e[1] * e[1]) + (e[2] * e[2] + e[3] * e[3])
                mean = jnp.sum(s) * inv_d
                var = jnp.sum(sq) * inv_d - mean * mean
                rstd = _fast_rsqrt(var + eps)
                mv = jnp.full((LANES,), mean, jnp.float32)
                rv = jnp.full((LANES,), rstd, jnp.float32)
                for j in range(NVREG):
                    outb_v[b, i, pl.ds(LANES * j, LANES)] = (
                        (e[j] - mv) * rv * g_regs[j] + b_regs[j])
                return carry

            lax.fori_loop(0, CHUNK, tok, jnp.int32(0))

        gather_start(0, 0)
        gather_start(1, 1)
        for b in range(NBUF):
            gather_wait(b, b)
            compute_chunk(b, b)
            store_start(b, b)
            gather_start(b + NBUF, b)

        def outer(k, carry):
            n = k * NBUF
            for b in range(NBUF):
                nb = n + b
                store_wait(nb - NBUF, b)
                gather_wait(nb, b)
                compute_chunk(nb, b)
                store_start(nb, b)
                gather_start(nb + NBUF, b)
            return carry

        lax.fori_loop(1, NCH // NBUF - 1, outer, jnp.int32(0))

        for b in range(NBUF):
            nb = NCH - NBUF + b
            store_wait(nb - NBUF, b)
            gather_wait(nb, b)
            compute_chunk(nb, b)
            store_start(nb, b)
        for b in range(NBUF):
            store_wait(NCH - NBUF + b, b)

    return emb_ln


def kernel(input_ids, token_table, pos_table, gamma, beta):
    B, L = input_ids.shape
    D = token_table.shape[1]
    NW = 32
    N = B * L
    NCH = N // (NW * CHUNK)

    ids = input_ids.astype(jnp.int32).reshape(NW, NCH, CHUNK)
    emb_ln = _make_kernel(B, L, D, NW)
    out = emb_ln(ids, token_table, pos_table, gamma, beta)
    return out.reshape(B, L, D)


# speedup vs baseline: 1.0301x; 1.0301x over previous
import functools

import jax
import jax.numpy as jnp
from jax import lax
from jax.experimental import pallas as pl
from jax.experimental.pallas import tpu as pltpu
from jax.experimental.pallas import tpu_sc as plsc

LANES = 16
CHUNK = 128
NBUF = 2


def _fast_rsqrt(x):
    xi = lax.bitcast_convert_type(x, jnp.int32)
    yi = jnp.int32(0x5F3759DF) - lax.shift_right_arithmetic(xi, 1)
    y = lax.bitcast_convert_type(yi, jnp.float32)
    xh = x * jnp.float32(0.5)
    for _ in range(3):
        y = y * (jnp.float32(1.5) - xh * y * y)
    return y


def _make_kernel(B, L, D, NW):
    N = B * L
    assert N % (NW * CHUNK) == 0
    NCH = N // (NW * CHUNK)
    PER_W = NCH * CHUNK
    assert PER_W % L == 0
    NVREG = D // LANES
    POS_ROWS = 2 * L

    mesh = plsc.VectorSubcoreMesh(core_axis_name="c", subcore_axis_name="s")

    @functools.partial(
        pl.kernel,
        mesh=mesh,
        compiler_params=pltpu.CompilerParams(
            needs_layout_passes=False, use_tc_tiling_on_sc=False),
        out_type=jax.ShapeDtypeStruct((N, D), jnp.float32),
        scratch_types=[
            pltpu.VMEM((NCH, CHUNK), jnp.int32),
            pltpu.VMEM((POS_ROWS, D), jnp.float32),
            pltpu.VMEM((NBUF, CHUNK, D), jnp.float32),
            pltpu.VMEM((NBUF, CHUNK, D), jnp.float32),
            pltpu.VMEM((D,), jnp.float32),
            pltpu.VMEM((D,), jnp.float32),
            pltpu.SemaphoreType.DMA,
            pltpu.SemaphoreType.DMA,
            pltpu.SemaphoreType.DMA,
            pltpu.SemaphoreType.DMA,
        ],
    )
    def emb_ln(ids_hbm, tok_hbm, pos_hbm, gamma_hbm, beta_hbm, out_hbm,
               idx_v, pos_v, rows_v, outb_v, g_v, b_v,
               gsem0, gsem1, ssem0, ssem1):
        gsems = (gsem0, gsem1)
        ssems = (ssem0, ssem1)
        wid = lax.axis_index("s") * 2 + lax.axis_index("c")
        wbase = wid * PER_W

        pltpu.sync_copy(ids_hbm.at[wid], idx_v)
        pltpu.sync_copy(pos_hbm.at[pl.ds(0, L)], pos_v.at[pl.ds(0, L)])
        pltpu.sync_copy(pos_hbm.at[pl.ds(0, L)], pos_v.at[pl.ds(L, L)])
        pltpu.sync_copy(gamma_hbm, g_v)
        pltpu.sync_copy(beta_hbm, b_v)

        g_regs = [g_v[pl.ds(LANES * j, LANES)] for j in range(NVREG)]
        b_regs = [b_v[pl.ds(LANES * j, LANES)] for j in range(NVREG)]

        inv_d = jnp.float32(1.0 / D)
        eps = jnp.float32(1e-12)

        def gather_start(n, b):
            pltpu.async_copy(tok_hbm.at[idx_v.at[n]], rows_v.at[b], gsems[b])

        def gather_wait(n, b):
            pltpu.make_async_copy(
                tok_hbm.at[idx_v.at[n]], rows_v.at[b], gsems[b]
            ).wait()

        def store_start(n, b):
            pltpu.async_copy(
                outb_v.at[b], out_hbm.at[pl.ds(wbase + n * CHUNK, CHUNK)],
                ssems[b])

        def store_wait(n, b):
            pltpu.make_async_copy(
                outb_v.at[b], out_hbm.at[pl.ds(wbase + n * CHUNK, CHUNK)],
                ssems[b]).wait()

        UNROLL = 8

        def compute_chunk(n, b):
            p0 = lax.rem(n * CHUNK, L)

            def one_token(i):
                r = [rows_v[b, i, pl.ds(LANES * j, LANES)]
                     for j in range(NVREG)]
                p = [pos_v[p0 + i, pl.ds(LANES * j, LANES)]
                     for j in range(NVREG)]
                e = [r[j] + p[j] for j in range(NVREG)]
                s = (e[0] + e[1]) + (e[2] + e[3])
                sq = (e[0] * e[0] + e[1] * e[1]) + (e[2] * e[2] + e[3] * e[3])
                mean = jnp.sum(s) * inv_d
                var = jnp.sum(sq) * inv_d - mean * mean
                rstd = _fast_rsqrt(var + eps)
                mv = jnp.full((LANES,), mean, jnp.float32)
                rv = jnp.full((LANES,), rstd, jnp.float32)
                for j in range(NVREG):
                    outb_v[b, i, pl.ds(LANES * j, LANES)] = (
                        (e[j] - mv) * rv * g_regs[j] + b_regs[j])

            def tok(i, carry):
                base = i * UNROLL
                for u in range(UNROLL):
                    one_token(base + u)
                return carry

            lax.fori_loop(0, CHUNK // UNROLL, tok, jnp.int32(0))

        gather_start(0, 0)
        gather_start(1, 1)
        for b in range(NBUF):
            gather_wait(b, b)
            compute_chunk(b, b)
            store_start(b, b)
            gather_start(b + NBUF, b)

        def outer(k, carry):
            n = k * NBUF
            for b in range(NBUF):
                nb = n + b
                store_wait(nb - NBUF, b)
                gather_wait(nb, b)
                compute_chunk(nb, b)
                store_start(nb, b)
                gather_start(nb + NBUF, b)
            return carry

        lax.fori_loop(1, NCH // NBUF - 1, outer, jnp.int32(0))

        for b in range(NBUF):
            nb = NCH - NBUF + b
            store_wait(nb - NBUF, b)
            gather_wait(nb, b)
            compute_chunk(nb, b)
            store_start(nb, b)
        for b in range(NBUF):
            store_wait(NCH - NBUF + b, b)

    return emb_ln


def kernel(input_ids, token_table, pos_table, gamma, beta):
    B, L = input_ids.shape
    D = token_table.shape[1]
    NW = 32
    N = B * L
    NCH = N // (NW * CHUNK)

    ids = input_ids.astype(jnp.int32).reshape(NW, NCH, CHUNK)
    emb_ln = _make_kernel(B, L, D, NW)
    out = emb_ln(ids, token_table, pos_table, gamma, beta)
    return out.reshape(B, L, D)
